# baseline (speedup 1.0000x reference)
import functools

import jax
import jax.numpy as jnp
from jax import lax
from jax.experimental import pallas as pl
from jax.experimental.pallas import tpu as pltpu
from jax.experimental.pallas import tpu_sc as plsc

N = 10000
E = 320000
D = 128
C = 16
W = D + C

NC = 2
NS = 16
NW = NC * NS
K = 80
STEPS = 125
NP_ = 10240
ROWS_PER_TILE = NP_ // NS


def _onehot_body(logits_ref, h_ref, hp_ref, p_ref, z0_ref):
    lg = logits_ref[...]
    m = jnp.max(lg, axis=1, keepdims=True)
    col = lax.broadcasted_iota(jnp.int32, lg.shape, 1)
    idx = jnp.min(jnp.where(lg == m, col, C), axis=1, keepdims=True)
    p = (col == idx).astype(jnp.float32)
    p_ref[...] = p
    hp_ref[:, :D] = h_ref[...]
    hp_ref[:, D:] = p
    z0_ref[...] = jnp.zeros((NP_, W), jnp.float32)


def _onehot_pred(logits, h):
    return pl.pallas_call(
        _onehot_body,
        out_shape=(
            jax.ShapeDtypeStruct((N, W), jnp.float32),
            jax.ShapeDtypeStruct((N, C), jnp.float32),
            jax.ShapeDtypeStruct((NP_, W), jnp.float32),
        ),
    )(logits, h)


def _sc_body(src_hbm, dst_hbm, hp_hbm, z_hbm,
             cnts_out, agg_out,
             srcb, dst_v, row_v, acc_sh,
             idx_sem, g_sem, s_sem):
    c = lax.axis_index("c")
    s = lax.axis_index("s")
    wid = s * NC + c

    r0 = s * ROWS_PER_TILE
    pltpu.sync_copy(z_hbm.at[pl.ds(r0, ROWS_PER_TILE)],
                    acc_sh.at[pl.ds(r0, ROWS_PER_TILE)])

    pltpu.sync_copy(dst_hbm.at[wid], dst_v)
    plsc.subcore_barrier()

    def issue_srcidx(i, b):
        pltpu.async_copy(src_hbm.at[wid, lax.rem(i, STEPS)], srcb.at[b],
                         idx_sem)

    def wait_srcidx(b):
        pltpu.make_async_copy(src_hbm.at[wid, 0], srcb.at[b],
                              idx_sem).wait()

    def issue_gather(b):
        pltpu.async_copy(hp_hbm.at[srcb.at[b]], row_v.at[b], g_sem)

    def wait_gather(b):
        pltpu.make_async_copy(hp_hbm.at[srcb.at[b]], row_v.at[b],
                              g_sem).wait()

    def issue_scatter(i, b):
        pltpu.async_copy(row_v.at[b], acc_sh.at[dst_v.at[i]], s_sem,
                         add=True)

    def wait_scatter(b):
        pltpu.make_async_copy(row_v.at[b], acc_sh.at[pl.ds(0, K)],
                              s_sem).wait()

    issue_srcidx(0, 0)
    wait_srcidx(0)
    issue_gather(0)
    issue_srcidx(1, 1)

    def step(g, carry):
        i0 = 2 * g
        i1 = i0 + 1
        wait_srcidx(1)
        issue_gather(1)
        wait_gather(0)
        issue_scatter(i0, 0)
        issue_srcidx(i0 + 2, 0)
        wait_gather(1)
        issue_scatter(i1, 1)
        issue_srcidx(i1 + 2, 1)
        wait_scatter(0)
        wait_srcidx(0)
        issue_gather(0)
        wait_scatter(1)
        return carry

    lax.fori_loop(0, (STEPS - 1) // 2, step, 0)
    wait_gather(0)
    issue_scatter(STEPS - 1, 0)
    wait_scatter(0)
    wait_srcidx(1)
    plsc.subcore_barrier()

    pltpu.sync_copy(acc_sh.at[pl.ds(r0, ROWS_PER_TILE), pl.ds(0, D)],
                    agg_out.at[c, pl.ds(r0, ROWS_PER_TILE)])
    pltpu.sync_copy(acc_sh.at[pl.ds(r0, ROWS_PER_TILE), pl.ds(D, C)],
                    cnts_out.at[c, pl.ds(r0, ROWS_PER_TILE)])


def _sc_aggregate(src, dst, hp, z):
    mesh = plsc.VectorSubcoreMesh(core_axis_name="c", subcore_axis_name="s")
    f = pl.kernel(
        _sc_body,
        out_type=(
            jax.ShapeDtypeStruct((NC, NP_, C), jnp.float32),
            jax.ShapeDtypeStruct((NC, NP_, D), jnp.float32),
        ),
        mesh=mesh,
        scratch_types=[
            pltpu.VMEM((2, K), jnp.int32),
            pltpu.VMEM((STEPS, K), jnp.int32),
            pltpu.VMEM((2, K, W), jnp.float32),
            pltpu.VMEM_SHARED((NP_, W), jnp.float32),
            pltpu.SemaphoreType.DMA,
            pltpu.SemaphoreType.DMA,
            pltpu.SemaphoreType.DMA,
        ],
        compiler_params=pltpu.CompilerParams(use_tc_tiling_on_sc=False),
    )
    return f(src.reshape(NW, STEPS, K), dst.reshape(NW, STEPS, K), hp, z)


BLK = 1000


def _c_body(cnts2_ref, p_ref, oldz_ref, t1_ref, t2_ref,
            h_ref, a0_ref, a1_ref, z_ref, out_ref, gate_sc):
    i = pl.program_id(0)

    @pl.when(i == 0)
    def _():
        cnts = cnts2_ref[0, :N, :] + cnts2_ref[1, :N, :]
        p = p_ref[...]
        f1 = jnp.sum(cnts * p, axis=1, keepdims=True)
        cc = jnp.maximum(cnts, 1e-5)
        f2 = -jnp.sum(cc * jnp.log(cc), axis=1, keepdims=True)

        def _ln(x):
            mu = jnp.mean(x)
            var = jnp.mean((x - mu) ** 2)
            return (x - mu) / jnp.sqrt(var + 1e-5)

        t1 = t1_ref[0, 0]
        t2 = t2_ref[0, 0]
        z = jax.nn.sigmoid(t1 - _ln(f1)) * jax.nn.sigmoid(t2 - _ln(f2))
        z_ref[...] = z
        gate_sc[...] = jnp.minimum(oldz_ref[...], z)

    @pl.when(i > 0)
    def _():
        agg = jax.nn.relu(a0_ref[0] + a1_ref[0])
        gate = gate_sc[pl.ds((i - 1) * BLK, BLK), :]
        out_ref[...] = h_ref[...] + gate * agg


def _gate_update(cnts2, p, old_z, tau_1, tau_2, h, agg2):
    nb = N // BLK
    blk = lambda i: jnp.maximum(i - 1, 0)
    const2 = lambda i: (0, 0)
    spec = pl.BlockSpec((BLK, D), lambda i: (blk(i), 0))
    a0spec = pl.BlockSpec((1, BLK, D), lambda i: (0, blk(i), 0))
    a1spec = pl.BlockSpec((1, BLK, D), lambda i: (1, blk(i), 0))
    return pl.pallas_call(
        _c_body,
        grid=(nb + 1,),
        in_specs=[
            pl.BlockSpec((NC, NP_, C), lambda i: (0, 0, 0)),
            pl.BlockSpec((N, C), const2),
            pl.BlockSpec((N, 1), const2),
            pl.BlockSpec((1, 1), const2),
            pl.BlockSpec((1, 1), const2),
            spec,
            a0spec,
            a1spec,
        ],
        out_specs=[
            pl.BlockSpec((N, 1), const2),
            spec,
        ],
        out_shape=(
            jax.ShapeDtypeStruct((N, 1), jnp.float32),
            jax.ShapeDtypeStruct((N, D), jnp.float32),
        ),
        scratch_shapes=[pltpu.VMEM((N, 1), jnp.float32)],
    )(cnts2, p, old_z, tau_1, tau_2, h, agg2, agg2)


def kernel(h, logits, old_z, edge_index, tau_1, tau_2):
    src = edge_index[0].astype(jnp.int32)
    dst = edge_index[1].astype(jnp.int32)

    hp, p, z0 = _onehot_pred(logits, h)
    cnts2, agg2 = _sc_aggregate(src, dst, hp, z0)
    z, new_h = _gate_update(cnts2, p, old_z.reshape(N, 1),
                            tau_1.reshape(1, 1), tau_2.reshape(1, 1),
                            h, agg2)
    return (new_h, z.reshape(N))


# speedup vs baseline: 1.0752x; 1.0752x over previous
import functools

import jax
import jax.numpy as jnp
from jax import lax
from jax.experimental import pallas as pl
from jax.experimental.pallas import tpu as pltpu
from jax.experimental.pallas import tpu_sc as plsc

N = 10000
E = 320000
D = 128
C = 16

NC = 2
NS = 16
NW = NC * NS
K = 80
STEPS = 125
EP = NW * STEPS * K
NP_ = 10240
ROWS_PER_TILE = NP_ // NS


def _onehot_body(logits_ref, p_ref, zc_ref, zd_ref):
    lg = logits_ref[...]
    m = jnp.max(lg, axis=1, keepdims=True)
    col = lax.broadcasted_iota(jnp.int32, lg.shape, 1)
    idx = jnp.min(jnp.where(lg == m, col, C), axis=1, keepdims=True)
    p_ref[...] = (col == idx).astype(jnp.float32)
    zc_ref[...] = jnp.zeros((NP_, C), jnp.float32)
    zd_ref[...] = jnp.zeros((NP_, D), jnp.float32)


def _onehot_pred(logits):
    return pl.pallas_call(
        _onehot_body,
        out_shape=(
            jax.ShapeDtypeStruct((N, C), jnp.float32),
            jax.ShapeDtypeStruct((NP_, C), jnp.float32),
            jax.ShapeDtypeStruct((NP_, D), jnp.float32),
        ),
    )(logits)


def _sc_body(src_hbm, dst_hbm, p_hbm, h_hbm, zc_hbm, zd_hbm,
             cnts_out, agg_out,
             srcb, dst_v, oh_v, row_v, cnts_sh, agg_sh,
             idx_sem, goh_sem, grow_sem, soh_sem, srow_sem):
    c = lax.axis_index("c")
    s = lax.axis_index("s")
    wid = s * NC + c

    r0 = s * ROWS_PER_TILE
    pltpu.sync_copy(zc_hbm.at[pl.ds(r0, ROWS_PER_TILE)],
                    cnts_sh.at[pl.ds(r0, ROWS_PER_TILE)])
    pltpu.sync_copy(zd_hbm.at[pl.ds(r0, ROWS_PER_TILE)],
                    agg_sh.at[pl.ds(r0, ROWS_PER_TILE)])
    plsc.subcore_barrier()

    pltpu.sync_copy(dst_hbm.at[wid], dst_v)

    def issue_srcidx(i, b):
        pltpu.async_copy(src_hbm.at[wid, lax.rem(i, STEPS)], srcb.at[b],
                         idx_sem)

    def wait_srcidx(b):
        pltpu.make_async_copy(src_hbm.at[wid, 0], srcb.at[b],
                              idx_sem).wait()

    def issue_gathers(b):
        pltpu.async_copy(p_hbm.at[srcb.at[b]], oh_v.at[b], goh_sem)
        pltpu.async_copy(h_hbm.at[srcb.at[b]], row_v.at[b], grow_sem)

    def wait_gathers(b):
        pltpu.make_async_copy(p_hbm.at[srcb.at[b]], oh_v.at[b],
                              goh_sem).wait()
        pltpu.make_async_copy(h_hbm.at[srcb.at[b]], row_v.at[b],
                              grow_sem).wait()

    def issue_scatters(i, b):
        pltpu.async_copy(oh_v.at[b], cnts_sh.at[dst_v.at[i]], soh_sem,
                         add=True)
        pltpu.async_copy(row_v.at[b], agg_sh.at[dst_v.at[i]], srow_sem,
                         add=True)

    def wait_scatters(b):
        pltpu.make_async_copy(oh_v.at[b], cnts_sh.at[pl.ds(0, K)],
                              soh_sem).wait()
        pltpu.make_async_copy(row_v.at[b], agg_sh.at[pl.ds(0, K)],
                              srow_sem).wait()

    issue_srcidx(0, 0)
    wait_srcidx(0)
    issue_gathers(0)
    issue_srcidx(1, 1)

    def step(g, carry):
        i0 = 2 * g
        i1 = i0 + 1
        wait_srcidx(1)
        issue_gathers(1)
        wait_gathers(0)
        issue_scatters(i0, 0)
        issue_srcidx(i0 + 2, 0)
        wait_gathers(1)
        issue_scatters(i1, 1)
        issue_srcidx(i1 + 2, 1)
        wait_scatters(0)
        wait_srcidx(0)
        issue_gathers(0)
        wait_scatters(1)
        return carry

    lax.fori_loop(0, (STEPS - 1) // 2, step, 0)
    wait_gathers(0)
    issue_scatters(STEPS - 1, 0)
    wait_scatters(0)
    wait_srcidx(1)
    plsc.subcore_barrier()

    pltpu.sync_copy(cnts_sh.at[pl.ds(r0, ROWS_PER_TILE)],
                    cnts_out.at[c, pl.ds(r0, ROWS_PER_TILE)])
    pltpu.sync_copy(agg_sh.at[pl.ds(r0, ROWS_PER_TILE)],
                    agg_out.at[c, pl.ds(r0, ROWS_PER_TILE)])


def _sc_aggregate(src, dst, p, h, zc, zd):
    mesh = plsc.VectorSubcoreMesh(core_axis_name="c", subcore_axis_name="s")
    f = pl.kernel(
        _sc_body,
        out_type=(
            jax.ShapeDtypeStruct((NC, NP_, C), jnp.float32),
            jax.ShapeDtypeStruct((NC, NP_, D), jnp.float32),
        ),
        mesh=mesh,
        scratch_types=[
            pltpu.VMEM((2, K), jnp.int32),
            pltpu.VMEM((STEPS, K), jnp.int32),
            pltpu.VMEM((2, K, C), jnp.float32),
            pltpu.VMEM((2, K, D), jnp.float32),
            pltpu.VMEM_SHARED((NP_, C), jnp.float32),
            pltpu.VMEM_SHARED((NP_, D), jnp.float32),
            pltpu.SemaphoreType.DMA,
            pltpu.SemaphoreType.DMA,
            pltpu.SemaphoreType.DMA,
            pltpu.SemaphoreType.DMA,
            pltpu.SemaphoreType.DMA,
        ],
        compiler_params=pltpu.CompilerParams(use_tc_tiling_on_sc=False),
    )
    return f(src.reshape(NW, STEPS, K), dst.reshape(NW, STEPS, K),
             p, h, zc, zd)


def _gate_body(cnts2_ref, p_ref, oldz_ref, t1_ref, t2_ref, z_ref, gate_ref):
    cnts = cnts2_ref[0, :N, :] + cnts2_ref[1, :N, :]
    p = p_ref[...]
    f1 = jnp.sum(cnts * p, axis=1, keepdims=True)
    cc = jnp.maximum(cnts, 1e-5)
    f2 = -jnp.sum(cc * jnp.log(cc), axis=1, keepdims=True)

    def _ln(x):
        mu = jnp.mean(x)
        var = jnp.mean((x - mu) ** 2)
        return (x - mu) / jnp.sqrt(var + 1e-5)

    nf1 = _ln(f1)
    nf2 = _ln(f2)
    t1 = t1_ref[0, 0]
    t2 = t2_ref[0, 0]
    z = jax.nn.sigmoid(t1 - nf1) * jax.nn.sigmoid(t2 - nf2)
    z_ref[...] = z
    gate_ref[...] = jnp.minimum(oldz_ref[...], z)


def _gates(cnts2, p, old_z, tau_1, tau_2):
    return pl.pallas_call(
        _gate_body,
        out_shape=(
            jax.ShapeDtypeStruct((N, 1), jnp.float32),
            jax.ShapeDtypeStruct((N, 1), jnp.float32),
        ),
    )(cnts2, p, old_z, tau_1, tau_2)


BLK = 1000


def _update_body(h_ref, a0_ref, a1_ref, gate_ref, out_ref):
    agg = jax.nn.relu(a0_ref[0] + a1_ref[0])
    out_ref[...] = h_ref[...] + gate_ref[...] * agg


def _update(h, agg2, gate):
    grid = (N // BLK,)
    spec = pl.BlockSpec((BLK, D), lambda i: (i, 0))
    a0spec = pl.BlockSpec((1, BLK, D), lambda i: (0, i, 0))
    a1spec = pl.BlockSpec((1, BLK, D), lambda i: (1, i, 0))
    gspec = pl.BlockSpec((BLK, 1), lambda i: (i, 0))
    return pl.pallas_call(
        _update_body,
        grid=grid,
        in_specs=[spec, a0spec, a1spec, gspec],
        out_specs=spec,
        out_shape=jax.ShapeDtypeStruct((N, D), jnp.float32),
    )(h, agg2, agg2, gate)


def kernel(h, logits, old_z, edge_index, tau_1, tau_2):
    src = edge_index[0].astype(jnp.int32)
    dst = edge_index[1].astype(jnp.int32)

    p, zc, zd = _onehot_pred(logits)
    cnts2, agg2 = _sc_aggregate(src, dst, p, h, zc, zd)

    z, gate = _gates(cnts2, p, old_z.reshape(N, 1),
                     tau_1.reshape(1, 1), tau_2.reshape(1, 1))
    new_h = _update(h, agg2, gate)
    return (new_h, z.reshape(N))
